# baseline (speedup 1.0000x reference)
import jax
import jax.numpy as jnp
from jax import lax
from jax.experimental import pallas as pl
from jax.experimental.pallas import tpu as pltpu
from jax.experimental.pallas import tpu_sc as plsc

_BATCH = 16384
_DIM = 3
_NC, _NS = 2, 16
_NW = _NC * _NS
_CHUNK = 128
_CPW = _BATCH // (_NW * _CHUNK)
_NTAB = 3
_SUB = _CHUNK // 16


def _body(idx_hbm, w0, w1, w2, out_hbm, idx_v, fidx_v, vals_v, shared, red_v,
          out_v, sem):
    c = lax.axis_index("c")
    s = lax.axis_index("s")
    w = c * _NS + s

    pltpu.sync_copy(idx_hbm.at[pl.ds(w * _CPW, _CPW)], idx_v)

    for j in range(_CPW):
        for cc in range(_SUB):
            v3 = idx_v[j, pl.ds(cc * 16, 16)] * 3
            for d in range(_DIM):
                fidx_v[d * _CPW + j, pl.ds(cc * 16, 16)] = v3 + d

    copies = []
    for t, tbl in enumerate((w0, w1, w2)):
        for dj in range(_DIM * _CPW):
            copies.append(
                pltpu.async_copy(
                    tbl.at[fidx_v.at[dj]], vals_v.at[t * _DIM * _CPW + dj], sem
                )
            )
    for cp in copies:
        cp.wait()

    accs = [jnp.zeros((16,), jnp.float32) for _ in range(_NTAB * _DIM)]
    for t in range(_NTAB):
        for d in range(_DIM):
            for j in range(_CPW):
                row = t * _DIM * _CPW + d * _CPW + j
                for cc in range(_SUB):
                    accs[t * _DIM + d] = (
                        accs[t * _DIM + d] + vals_v[row, pl.ds(cc * 16, 16)]
                    )

    iota = lax.iota(jnp.int32, 16)

    _dnums = lax.GatherDimensionNumbers(
        offset_dims=(), collapsed_slice_dims=(0,), start_index_map=(0,)
    )

    def _shuffle(v, idx16):
        return lax.gather(
            v,
            idx16[:, None],
            _dnums,
            slice_sizes=(1,),
            mode=lax.GatherScatterMode.PROMISE_IN_BOUNDS,
        )

    def _lane_sum(v):
        for sh in (1, 2, 4, 8):
            v = v + _shuffle(v, jnp.bitwise_xor(iota, sh))
        return v

    part = jnp.zeros((16,), jnp.float32)
    for k in range(_NTAB * _DIM):
        part = jnp.where(iota == k, _lane_sum(accs[k]), part)
    scale = jnp.where(
        iota < 3, 5.0, jnp.where(iota < 6, 10.0, jnp.where(iota < 9, 6.0, 0.0))
    ).astype(jnp.float32)
    out_v[...] = part * scale

    pltpu.sync_copy(out_v, out_hbm.at[w])


_sc_call = pl.kernel(
    _body,
    out_type=jax.ShapeDtypeStruct((_NW, 16), jnp.float32),
    mesh=plsc.VectorSubcoreMesh(core_axis_name="c", subcore_axis_name="s"),
    scratch_types=[
        pltpu.VMEM((_CPW, _CHUNK), jnp.int32),
        pltpu.VMEM((_DIM * _CPW, _CHUNK), jnp.int32),
        pltpu.VMEM((_NTAB * _DIM * _CPW, _CHUNK), jnp.float32),
        pltpu.VMEM_SHARED((_NS, 16), jnp.float32),
        pltpu.VMEM((_NS, 16), jnp.float32),
        pltpu.VMEM((16,), jnp.float32),
        pltpu.SemaphoreType.DMA,
    ],
)


@jax.jit
def kernel(e_input, W0, W1, W2):
    idx = e_input.reshape(_NW * _CPW, _CHUNK).astype(jnp.int32)
    out = _sc_call(
        idx, W0.reshape(-1), W1.reshape(-1), W2.reshape(-1)
    )
    return out.sum(axis=0)[: _NTAB * _DIM].reshape(_NTAB, _DIM)
